# Initial kernel scaffold; baseline (speedup 1.0000x reference)
import jax
import jax.numpy as jnp
from jax.experimental import pallas as pl


def kernel(x, W):
    raise NotImplementedError("write your pallas kernel here")


# speedup vs baseline: 1.1751x; 1.1751x over previous
import jax
import jax.numpy as jnp
from jax.experimental import pallas as pl

NTOK = 32768
HIDDEN = 4096
NUM_EXPERTS = 64
TOP_K = 8
BLK = 512


def _router_block(x_ref, w_ref, probs_ref, idx_ref):
    x = x_ref[...]
    w = w_ref[...]
    logits = jax.lax.dot_general(
        x, w, (((1,), (1,)), ((), ())),
        preferred_element_type=jnp.float32,
        precision=jax.lax.Precision.DEFAULT,
    )
    m = jnp.max(logits, axis=-1, keepdims=True)
    e = jnp.exp(logits - m)
    p = e / jnp.sum(e, axis=-1, keepdims=True)
    probs_ref[...] = p

    iota_e = jax.lax.broadcasted_iota(jnp.int32, (BLK, NUM_EXPERTS), 1)
    iota_k = jax.lax.broadcasted_iota(jnp.int32, (BLK, TOP_K), 1)
    work = p
    idx_out = jnp.zeros((BLK, TOP_K), dtype=jnp.int32)
    for k in range(TOP_K):
        mx = jnp.max(work, axis=-1, keepdims=True)
        amax = jnp.min(
            jnp.where(work == mx, iota_e, NUM_EXPERTS), axis=-1, keepdims=True
        )
        idx_out = idx_out + jnp.where(iota_k == k, amax, 0)
        work = jnp.where(iota_e == amax, -1.0, work)
    idx_ref[...] = idx_out


def kernel(x, W):
    grid = (NTOK // BLK,)
    probs, idx = pl.pallas_call(
        _router_block,
        grid=grid,
        in_specs=[
            pl.BlockSpec((BLK, HIDDEN), lambda i: (i, 0)),
            pl.BlockSpec((NUM_EXPERTS, HIDDEN), lambda i: (0, 0)),
        ],
        out_specs=[
            pl.BlockSpec((BLK, NUM_EXPERTS), lambda i: (i, 0)),
            pl.BlockSpec((BLK, TOP_K), lambda i: (i, 0)),
        ],
        out_shape=[
            jax.ShapeDtypeStruct((NTOK, NUM_EXPERTS), jnp.float32),
            jax.ShapeDtypeStruct((NTOK, TOP_K), jnp.int32),
        ],
    )(x, W)
    return (probs, idx)


# speedup vs baseline: 1.4316x; 1.2183x over previous
import jax
import jax.numpy as jnp
from jax.experimental import pallas as pl

NTOK = 32768
HIDDEN = 4096
NUM_EXPERTS = 64
TOP_K = 8
BLK = 512


def _router_block(x_ref, w_ref, probs_ref, idx_ref):
    x = x_ref[...]
    w = w_ref[...]
    logits = jax.lax.dot_general(
        x, w, (((1,), (1,)), ((), ())),
        preferred_element_type=jnp.float32,
        precision=jax.lax.Precision.DEFAULT,
    )
    m = jnp.max(logits, axis=-1, keepdims=True)
    e = jnp.exp(logits - m)
    p = e / jnp.sum(e, axis=-1, keepdims=True)
    probs_ref[...] = p

    iota_e = jax.lax.broadcasted_iota(jnp.int32, (BLK, NUM_EXPERTS), 1)
    iota_k = jax.lax.broadcasted_iota(jnp.int32, (BLK, TOP_K), 1)
    p_bits = jax.lax.bitcast_convert_type(p, jnp.int32)
    key = jax.lax.bitcast_convert_type(
        (p_bits & jnp.int32(~63)) | (63 - iota_e), jnp.float32
    )
    idx_out = jnp.zeros((BLK, TOP_K), dtype=jnp.int32)
    for k in range(TOP_K):
        mx = jnp.max(key, axis=-1, keepdims=True)
        amax = 63 - (jax.lax.bitcast_convert_type(mx, jnp.int32) & 63)
        idx_out = idx_out + jnp.where(iota_k == k, amax, 0)
        key = jnp.where(key == mx, -1.0, key)
    idx_ref[...] = idx_out


def kernel(x, W):
    grid = (NTOK // BLK,)
    probs, idx = pl.pallas_call(
        _router_block,
        grid=grid,
        in_specs=[
            pl.BlockSpec((BLK, HIDDEN), lambda i: (i, 0)),
            pl.BlockSpec((NUM_EXPERTS, HIDDEN), lambda i: (0, 0)),
        ],
        out_specs=[
            pl.BlockSpec((BLK, NUM_EXPERTS), lambda i: (i, 0)),
            pl.BlockSpec((BLK, TOP_K), lambda i: (i, 0)),
        ],
        out_shape=[
            jax.ShapeDtypeStruct((NTOK, NUM_EXPERTS), jnp.float32),
            jax.ShapeDtypeStruct((NTOK, TOP_K), jnp.int32),
        ],
    )(x, W)
    return (probs, idx)


# speedup vs baseline: 1.5666x; 1.0944x over previous
import jax
import jax.numpy as jnp
from jax.experimental import pallas as pl

NTOK = 32768
HIDDEN = 4096
NUM_EXPERTS = 64
TOP_K = 8
BLK = 1024


def _router_block(x_ref, w_ref, probs_ref, idx_ref):
    x = x_ref[...]
    w = w_ref[...]
    logits = jax.lax.dot_general(
        x, w, (((1,), (1,)), ((), ())),
        preferred_element_type=jnp.float32,
        precision=jax.lax.Precision.DEFAULT,
    )
    m = jnp.max(logits, axis=-1, keepdims=True)
    e = jnp.exp(logits - m)
    p = e / jnp.sum(e, axis=-1, keepdims=True)
    probs_ref[...] = p

    iota_e = jax.lax.broadcasted_iota(jnp.int32, (BLK, NUM_EXPERTS), 1)
    iota_k = jax.lax.broadcasted_iota(jnp.int32, (BLK, TOP_K), 1)
    p_bits = jax.lax.bitcast_convert_type(p, jnp.int32)
    key = jax.lax.bitcast_convert_type(
        (p_bits & jnp.int32(~63)) | (63 - iota_e), jnp.float32
    )
    idx_out = jnp.zeros((BLK, TOP_K), dtype=jnp.int32)
    for k in range(TOP_K):
        mx = jnp.max(key, axis=-1, keepdims=True)
        amax = 63 - (jax.lax.bitcast_convert_type(mx, jnp.int32) & 63)
        idx_out = idx_out + jnp.where(iota_k == k, amax, 0)
        key = jnp.where(key == mx, -1.0, key)
    idx_ref[...] = idx_out


def kernel(x, W):
    grid = (NTOK // BLK,)
    probs, idx = pl.pallas_call(
        _router_block,
        grid=grid,
        in_specs=[
            pl.BlockSpec((BLK, HIDDEN), lambda i: (i, 0)),
            pl.BlockSpec((NUM_EXPERTS, HIDDEN), lambda i: (0, 0)),
        ],
        out_specs=[
            pl.BlockSpec((BLK, NUM_EXPERTS), lambda i: (i, 0)),
            pl.BlockSpec((BLK, TOP_K), lambda i: (i, 0)),
        ],
        out_shape=[
            jax.ShapeDtypeStruct((NTOK, NUM_EXPERTS), jnp.float32),
            jax.ShapeDtypeStruct((NTOK, TOP_K), jnp.int32),
        ],
    )(x, W)
    return (probs, idx)
